# baseline (speedup 1.0000x reference)
import functools

import jax
import jax.numpy as jnp
from jax import lax
from jax.experimental import pallas as pl
from jax.experimental.pallas import tpu as pltpu

N_NODES = 10000
N_EDGES = 160000
MAX_PAIRS = 3000000


def _graph_norm(x, w, b, ms, eps=1e-5):
    mean = jnp.mean(x, axis=0)
    xc = x - ms * mean
    var = jnp.mean(xc * xc, axis=0)
    return w * xc / jnp.sqrt(var + eps) + b


def _join(A, B, n, pad):
    srcB = B[0]
    order = jnp.argsort(srcB, stable=True)
    counts = jnp.bincount(srcB, length=n)
    group_start = jnp.cumsum(counts) - counts
    mid = A[1]
    rep = counts[mid]
    total = rep.sum()
    a_idx = jnp.repeat(jnp.arange(A.shape[1]), rep, total_repeat_length=pad)
    cum = jnp.cumsum(rep) - rep
    seq = jnp.arange(pad) - jnp.repeat(cum, rep, total_repeat_length=pad)
    b_pos = jnp.clip(jnp.repeat(group_start[mid], rep, total_repeat_length=pad) + seq, 0, B.shape[1] - 1)
    b_idx = order[b_pos]
    valid = jnp.arange(pad) < total
    key = jnp.where(valid, A[0][a_idx] * n + B[1][b_idx], n * n)
    uk, inv = jnp.unique(key, return_inverse=True, size=pad, fill_value=n * n)
    return a_idx, b_idx, inv.reshape(-1), uk, valid


def _stats_body(eun_ref, c_ref, one_ref, w3_ref, wrow_ref, b3_ref, s1_ref, s2_ref):
    pid = pl.program_id(0)
    R = c_ref.shape[0]
    x3 = (jnp.dot(c_ref[...], w3_ref[...], preferred_element_type=jnp.float32)
          + one_ref[...] * wrow_ref[...] + b3_ref[...])
    gidx = pid * R + lax.broadcasted_iota(jnp.int32, (R, 1), 0)
    rowm = gidx < eun_ref[0]
    x3m = jnp.where(rowm, x3, 0.0)
    s1 = jnp.sum(x3m, axis=0, keepdims=True)
    s2 = jnp.sum(x3m * x3m, axis=0, keepdims=True)

    @pl.when(pid == 0)
    def _():
        s1_ref[...] = jnp.zeros_like(s1_ref)
        s2_ref[...] = jnp.zeros_like(s2_ref)

    s1_ref[...] += s1
    s2_ref[...] += s2


def _masked_stats(Cfull, onecol, Eu_n, m3_W, m3_b, block_rows):
    Eu, D = Cfull.shape
    grid = Eu // block_rows
    assert grid * block_rows == Eu
    s1, s2 = pl.pallas_call(
        _stats_body,
        grid=(grid,),
        in_specs=[
            pl.BlockSpec(memory_space=pltpu.SMEM),
            pl.BlockSpec((block_rows, D), lambda i: (i, 0)),
            pl.BlockSpec((block_rows, 1), lambda i: (i, 0)),
            pl.BlockSpec((D, D), lambda i: (0, 0)),
            pl.BlockSpec((1, D), lambda i: (0, 0)),
            pl.BlockSpec((1, D), lambda i: (0, 0)),
        ],
        out_specs=[
            pl.BlockSpec((1, D), lambda i: (0, 0)),
            pl.BlockSpec((1, D), lambda i: (0, 0)),
        ],
        out_shape=[
            jax.ShapeDtypeStruct((1, D), jnp.float32),
            jax.ShapeDtypeStruct((1, D), jnp.float32),
        ],
    )(
        jnp.asarray([Eu_n], jnp.int32) if not isinstance(Eu_n, jnp.ndarray) else Eu_n.reshape(1).astype(jnp.int32),
        Cfull,
        onecol,
        m3_W[:D],
        m3_W[D:D + 1],
        m3_b.reshape(1, D),
    )
    return s1[0], s2[0]


def kernel(x, ei, pos, emb, gcn1_W, gcn1_b, gn1_w, gn1_b, gn1_m, gcn2_W, gcn2_b,
           gn2_w, gn2_b, gn2_m, m1_W, m1_b, m2_W, m2_b, m3_W, m3_b, gn3_w,
           gn3_b, gn3_m, dir_W, dir_b):
    n = x.shape[0]
    m = ei.shape[1]
    sent = n * n

    src = jnp.concatenate([ei[0], jnp.arange(n)])
    dst = jnp.concatenate([ei[1], jnp.arange(n)])
    deg = jnp.bincount(dst, length=n).astype(jnp.float32)
    dis = 1.0 / jnp.sqrt(deg)
    norm = (dis[src] * dis[dst]).astype(jnp.float32)

    a_idx, b_idx, inv, uk, valid = _join(ei, ei, n, MAX_PAIRS)
    keys_e = ei[0] * n + ei[1]
    union = jnp.unique(jnp.concatenate([uk, keys_e]), size=MAX_PAIRS + m, fill_value=sent)
    Eu = union.shape[0]
    Eu_n = jnp.sum(union < sent)
    pos_c = jnp.searchsorted(union, uk)
    pos_e = jnp.searchsorted(union, keys_e)
    onecol = jnp.zeros((Eu,), jnp.float32).at[pos_e].add(1.0)[:, None]
    i_u = union // n
    j_u = union % n
    perm = jnp.argsort(jnp.where(union < sent, j_u * n + i_u, sent), stable=True)
    key_p = pos[:, 0] * n + pos[:, 1]
    idx = jnp.searchsorted(union, key_p)
    idx_c = jnp.minimum(idx, Eu - 1)
    pred = jnp.where(union[idx_c] == key_p, idx_c, Eu)

    h = emb[x]
    h1 = jax.ops.segment_sum(norm[:, None] * (h @ gcn1_W)[src], dst, num_segments=n) + gcn1_b
    h1 = jnp.maximum(_graph_norm(h1, gn1_w, gn1_b, gn1_m), 0.0)
    h2 = jax.ops.segment_sum(norm[:, None] * (h1 @ gcn2_W)[src], dst, num_segments=n) + gcn2_b
    h2 = jnp.maximum(_graph_norm(h2, gn2_w, gn2_b, gn2_m), 0.0)
    xx = h2[pos[:, 0]] * h2[pos[:, 1]]
    val = jnp.concatenate([h2[ei[0]], h2[ei[1]]], axis=1)
    xe = jnp.maximum(val @ m1_W + m1_b, 0.0)
    mul = jnp.maximum(val @ m2_W + m2_b, 0.0)

    sidx = pos_c[inv]
    prod = jnp.where(valid[:, None], xe[a_idx] * mul[b_idx], 0.0)
    Cfull = jnp.zeros((Eu, xe.shape[1]), jnp.float32).at[sidx].add(prod)

    s1, s2 = _masked_stats(Cfull, onecol, Eu_n, m3_W, m3_b, block_rows=4000)
    den = Eu_n.astype(jnp.float32)
    mean3 = s1 / den
    mm = gn3_m * mean3
    var3 = s2 / den - 2.0 * mm * (s1 / den) + mm * mm

    predc = jnp.minimum(pred, Eu - 1)
    permp = perm[predc]
    rows = jnp.concatenate([predc, permp])
    Crows = Cfull[rows]
    orows = onecol[rows]
    x3r = Crows @ m3_W[:-1] + orows * m3_W[-1:] + m3_b
    x3r = jnp.maximum(gn3_w * (x3r - mm) / jnp.sqrt(var3 + 1e-5) + gn3_b, 0.0)
    npos = pos.shape[0]
    xp = jnp.where((pred < Eu)[:, None], x3r[:npos] * x3r[npos:], 0.0)
    out = jnp.concatenate([xp, xx], axis=1) @ dir_W + dir_b
    return out


# speedup vs baseline: 1.1272x; 1.1272x over previous
import functools

import jax
import jax.numpy as jnp
from jax import lax
from jax.experimental import pallas as pl
from jax.experimental.pallas import tpu as pltpu

MAX_PAIRS = 3000000
BLK = 4000


def _graph_norm(x, w, b, ms, eps=1e-5):
    mean = jnp.mean(x, axis=0)
    xc = x - ms * mean
    var = jnp.mean(xc * xc, axis=0)
    return w * xc / jnp.sqrt(var + eps) + b


def _stats_body(eun_ref, c_ref, one_ref, w3_ref, wrow_ref, b3_ref, s1_ref, s2_ref):
    pid = pl.program_id(0)
    R = c_ref.shape[0]
    x3 = (jnp.dot(c_ref[...], w3_ref[...], preferred_element_type=jnp.float32)
          + one_ref[...] * wrow_ref[...] + b3_ref[...])
    gidx = pid * R + lax.broadcasted_iota(jnp.int32, (R, 1), 0)
    rowm = gidx < eun_ref[0]
    x3m = jnp.where(rowm, x3, 0.0)
    s1 = jnp.sum(x3m, axis=0, keepdims=True)
    s2 = jnp.sum(x3m * x3m, axis=0, keepdims=True)

    @pl.when(pid == 0)
    def _():
        s1_ref[...] = jnp.zeros_like(s1_ref)
        s2_ref[...] = jnp.zeros_like(s2_ref)

    s1_ref[...] += s1
    s2_ref[...] += s2


def _masked_stats(Cfull, onecol, Eu_n, m3_W, m3_b):
    Eup, D = Cfull.shape
    grid = Eup // BLK
    s1, s2 = pl.pallas_call(
        _stats_body,
        grid=(grid,),
        in_specs=[
            pl.BlockSpec(memory_space=pltpu.SMEM),
            pl.BlockSpec((BLK, D), lambda i: (i, 0)),
            pl.BlockSpec((BLK, 1), lambda i: (i, 0)),
            pl.BlockSpec((D, D), lambda i: (0, 0)),
            pl.BlockSpec((1, D), lambda i: (0, 0)),
            pl.BlockSpec((1, D), lambda i: (0, 0)),
        ],
        out_specs=[
            pl.BlockSpec((1, D), lambda i: (0, 0)),
            pl.BlockSpec((1, D), lambda i: (0, 0)),
        ],
        out_shape=[
            jax.ShapeDtypeStruct((1, D), jnp.float32),
            jax.ShapeDtypeStruct((1, D), jnp.float32),
        ],
    )(
        Eu_n.reshape(1).astype(jnp.int32),
        Cfull,
        onecol,
        m3_W[:D],
        m3_W[D:D + 1],
        m3_b.reshape(1, D),
    )
    return s1[0], s2[0]


def kernel(x, ei, pos, emb, gcn1_W, gcn1_b, gn1_w, gn1_b, gn1_m, gcn2_W, gcn2_b,
           gn2_w, gn2_b, gn2_m, m1_W, m1_b, m2_W, m2_b, m3_W, m3_b, gn3_w,
           gn3_b, gn3_m, dir_W, dir_b):
    n = x.shape[0]
    m = ei.shape[1]
    sent = n * n
    Eu = MAX_PAIRS + m
    Eup = ((Eu + BLK) // BLK) * BLK

    ei0 = ei[0]
    ei1 = ei[1]
    order = jnp.argsort(ei0, stable=True)
    counts = jnp.bincount(ei0, length=n)
    group_start = jnp.cumsum(counts) - counts
    rep = counts[ei1]
    total = rep.sum()
    cum = jnp.cumsum(rep) - rep
    z = jnp.full((MAX_PAIRS,), -1, jnp.int32).at[cum].max(
        jnp.arange(m, dtype=jnp.int32), mode='drop')
    a_idx = lax.cummax(z)
    cum_e = cum.astype(jnp.int32)
    gsm_e = group_start[ei1].astype(jnp.int32)
    i_n_e = (ei0 * n).astype(jnp.int32)
    parange = jnp.arange(MAX_PAIRS, dtype=jnp.int32)
    seq = parange - cum_e[a_idx]
    b_pos = jnp.clip(gsm_e[a_idx] + seq, 0, m - 1)
    b_idx = order[b_pos]
    valid = parange < total
    key = jnp.where(valid, i_n_e[a_idx] + ei1[b_idx], sent)

    keys_e = ei0 * n + ei1
    allkeys = jnp.concatenate([key, keys_e])
    ga_all = jnp.concatenate([jnp.where(valid, a_idx, m), jnp.full((m,), m, jnp.int32)])
    gb_all = jnp.concatenate([jnp.where(valid, b_idx, m), jnp.full((m,), m, jnp.int32)])
    sp = jnp.argsort(allkeys, stable=True)
    ks = allkeys[sp]
    ga = ga_all[sp]
    gb = gb_all[sp]
    isedge = (sp >= MAX_PAIRS).astype(jnp.float32)
    newk = jnp.concatenate([jnp.ones((1,), jnp.int32),
                            (ks[1:] != ks[:-1]).astype(jnp.int32)])
    rank = jnp.cumsum(newk) - 1
    n_uniq = rank[-1] + 1
    Eu_n = n_uniq - (ks[-1] == sent).astype(jnp.int32)
    union = jax.ops.segment_max(ks, rank, num_segments=Eu, indices_are_sorted=True)
    union = jnp.where(jnp.arange(Eu) < n_uniq, union, sent)
    onecol = jax.ops.segment_sum(isedge, rank, num_segments=Eup,
                                 indices_are_sorted=True)[:, None]
    i_u = union // n
    j_u = union % n
    perm = jnp.argsort(jnp.where(union < sent, j_u * n + i_u, sent), stable=True)
    key_p = pos[:, 0] * n + pos[:, 1]
    idxq = jnp.searchsorted(union, key_p)
    idx_c = jnp.minimum(idxq, Eu - 1)
    pred = jnp.where(union[idx_c] == key_p, idx_c, Eu)

    src = jnp.concatenate([ei0, jnp.arange(n)])
    dst = jnp.concatenate([ei1, jnp.arange(n)])
    deg = jnp.bincount(dst, length=n).astype(jnp.float32)
    dis = 1.0 / jnp.sqrt(deg)
    norm = (dis[src] * dis[dst]).astype(jnp.float32)
    h = emb[x]
    h1 = jax.ops.segment_sum(norm[:, None] * (h @ gcn1_W)[src], dst, num_segments=n) + gcn1_b
    h1 = jnp.maximum(_graph_norm(h1, gn1_w, gn1_b, gn1_m), 0.0)
    h2 = jax.ops.segment_sum(norm[:, None] * (h1 @ gcn2_W)[src], dst, num_segments=n) + gcn2_b
    h2 = jnp.maximum(_graph_norm(h2, gn2_w, gn2_b, gn2_m), 0.0)
    xx = h2[pos[:, 0]] * h2[pos[:, 1]]
    val = jnp.concatenate([h2[ei0], h2[ei1]], axis=1)
    xez = jnp.concatenate([jnp.maximum(val @ m1_W + m1_b, 0.0),
                           jnp.zeros((1, m1_W.shape[1]), jnp.float32)])
    mulz = jnp.concatenate([jnp.maximum(val @ m2_W + m2_b, 0.0),
                            jnp.zeros((1, m2_W.shape[1]), jnp.float32)])

    prod_s = xez[ga] * mulz[gb]
    Cfull = jax.ops.segment_sum(prod_s, rank, num_segments=Eup,
                                indices_are_sorted=True)

    s1, s2 = _masked_stats(Cfull, onecol, Eu_n, m3_W, m3_b)
    den = Eu_n.astype(jnp.float32)
    mean3 = s1 / den
    mm = gn3_m * mean3
    var3 = s2 / den - 2.0 * mm * (s1 / den) + mm * mm

    predc = jnp.minimum(pred, Eu - 1)
    permp = perm[predc]
    rows = jnp.concatenate([predc, permp])
    Crows = Cfull[rows]
    orows = onecol[rows]
    x3r = Crows @ m3_W[:-1] + orows * m3_W[-1:] + m3_b
    x3r = jnp.maximum(gn3_w * (x3r - mm) / jnp.sqrt(var3 + 1e-5) + gn3_b, 0.0)
    npos = pos.shape[0]
    xp = jnp.where((pred < Eu)[:, None], x3r[:npos] * x3r[npos:], 0.0)
    out = jnp.concatenate([xp, xx], axis=1) @ dir_W + dir_b
    return out


# speedup vs baseline: 11.2037x; 9.9395x over previous
import functools

import jax
import jax.numpy as jnp
from jax import lax
from jax.experimental import pallas as pl
from jax.experimental.pallas import tpu as pltpu

MAX_PAIRS = 3000000
BLK = 4000
SPAD = 2097152
RPAD = 1048576


def _graph_norm(x, w, b, ms, eps=1e-5):
    mean = jnp.mean(x, axis=0)
    xc = x - ms * mean
    var = jnp.mean(xc * xc, axis=0)
    return w * xc / jnp.sqrt(var + eps) + b


def _pad1(a, length, fill=0):
    return jnp.concatenate([a, jnp.full((length - a.shape[0],), fill, a.dtype)])


def _stats_body(eun_ref, c_ref, one_ref, w3_ref, wrow_ref, b3_ref, s1_ref, s2_ref):
    pid = pl.program_id(0)
    R = c_ref.shape[0]
    x3 = (jnp.dot(c_ref[...], w3_ref[...], preferred_element_type=jnp.float32)
          + one_ref[...] * wrow_ref[...] + b3_ref[...])
    gidx = pid * R + lax.broadcasted_iota(jnp.int32, (R, 1), 0)
    rowm = gidx < eun_ref[0]
    x3m = jnp.where(rowm, x3, 0.0)
    s1 = jnp.sum(x3m, axis=0, keepdims=True)
    s2 = jnp.sum(x3m * x3m, axis=0, keepdims=True)

    @pl.when(pid == 0)
    def _():
        s1_ref[...] = jnp.zeros_like(s1_ref)
        s2_ref[...] = jnp.zeros_like(s2_ref)

    s1_ref[...] += s1
    s2_ref[...] += s2


def _masked_stats(Cfull, onecol, Eu_n, m3_W, m3_b):
    Eup, D = Cfull.shape
    grid = Eup // BLK
    s1, s2 = pl.pallas_call(
        _stats_body,
        grid=(grid,),
        in_specs=[
            pl.BlockSpec(memory_space=pltpu.SMEM),
            pl.BlockSpec((BLK, D), lambda i: (i, 0)),
            pl.BlockSpec((BLK, 1), lambda i: (i, 0)),
            pl.BlockSpec((D, D), lambda i: (0, 0)),
            pl.BlockSpec((1, D), lambda i: (0, 0)),
            pl.BlockSpec((1, D), lambda i: (0, 0)),
        ],
        out_specs=[
            pl.BlockSpec((1, D), lambda i: (0, 0)),
            pl.BlockSpec((1, D), lambda i: (0, 0)),
        ],
        out_shape=[
            jax.ShapeDtypeStruct((1, D), jnp.float32),
            jax.ShapeDtypeStruct((1, D), jnp.float32),
        ],
    )(
        Eu_n.reshape(1).astype(jnp.int32),
        Cfull,
        onecol,
        m3_W[:D],
        m3_W[D:D + 1],
        m3_b.reshape(1, D),
    )
    return s1[0], s2[0]


def kernel(x, ei, pos, emb, gcn1_W, gcn1_b, gn1_w, gn1_b, gn1_m, gcn2_W, gcn2_b,
           gn2_w, gn2_b, gn2_m, m1_W, m1_b, m2_W, m2_b, m3_W, m3_b, gn3_w,
           gn3_b, gn3_m, dir_W, dir_b):
    n = x.shape[0]
    m = ei.shape[1]
    sent = n * n
    Eu = MAX_PAIRS + m
    Eup = ((Eu + BLK) // BLK) * BLK

    ei0 = ei[0]
    ei1 = ei[1]
    order = jnp.argsort(ei0, stable=True).astype(jnp.int32)
    counts = jnp.bincount(ei0, length=n)
    group_start = (jnp.cumsum(counts) - counts).astype(jnp.int32)
    rep = counts[ei1]
    total = rep.sum()
    cum = (jnp.cumsum(rep) - rep).astype(jnp.int32)
    z = jnp.full((MAX_PAIRS,), -1, jnp.int32).at[cum].max(
        jnp.arange(m, dtype=jnp.int32), mode='drop')
    a_idx = lax.cummax(z)
    delta_e = _pad1(group_start[ei1] - cum, SPAD)
    i_n_e = _pad1((ei0 * n).astype(jnp.int32), SPAD)
    packedord = _pad1(order * 10000 + ei1[order], SPAD)
    parange = jnp.arange(MAX_PAIRS, dtype=jnp.int32)
    b_pos = jnp.clip(delta_e[a_idx] + parange, 0, m - 1)
    pko = packedord[b_pos]
    b_idx = pko // 10000
    jdst = pko - b_idx * 10000
    valid = parange < total
    key = jnp.where(valid, i_n_e[a_idx] + jdst, sent)

    keys_e = ei0 * n + ei1
    allkeys = jnp.concatenate([key, keys_e])
    ga_all = jnp.concatenate([jnp.where(valid, a_idx, m),
                              jnp.full((m,), m, jnp.int32)])
    gb_all = jnp.concatenate([jnp.where(valid, b_idx, m),
                              jnp.full((m,), m, jnp.int32)])
    sp = jnp.argsort(allkeys, stable=True).astype(jnp.int32)
    ks = allkeys[sp]
    ga = ga_all[sp]
    gb = gb_all[sp]
    newk = jnp.concatenate([jnp.ones((1,), jnp.int32),
                            (ks[1:] != ks[:-1]).astype(jnp.int32)])
    rank = jnp.cumsum(newk) - 1
    n_uniq = rank[-1] + 1
    Eu_n = n_uniq - (ks[-1] == sent).astype(jnp.int32)

    pos_e = jnp.searchsorted(ks, keys_e).astype(jnp.int32)
    rank_e = rank[pos_e]
    onecol = jnp.zeros((Eup,), jnp.float32).at[rank_e].add(1.0)[:, None]

    key_p = pos[:, 0] * n + pos[:, 1]
    qpos = jnp.minimum(jnp.searchsorted(ks, key_p).astype(jnp.int32), Eu - 1)
    qhit = ks[qpos] == key_p
    pred = jnp.where(qhit, rank[qpos], Eu)

    tkeys = jnp.where(allkeys < sent, (allkeys % n) * n + allkeys // n, sent)
    tsp = jnp.argsort(tkeys, stable=True).astype(jnp.int32)
    tks = tkeys[tsp]
    tnew = jnp.concatenate([jnp.ones((1,), jnp.int32),
                            (tks[1:] != tks[:-1]).astype(jnp.int32)])
    trank = jnp.cumsum(tnew) - 1
    predc = jnp.minimum(pred, Eu - 1)
    tfirst = jnp.minimum(jnp.searchsorted(trank, predc).astype(jnp.int32), Eu - 1)
    partner_key = allkeys[tsp[tfirst]]
    ppos = jnp.minimum(jnp.searchsorted(ks, partner_key).astype(jnp.int32), Eu - 1)
    permp = rank[ppos]

    src = jnp.concatenate([ei0, jnp.arange(n)])
    dst = jnp.concatenate([ei1, jnp.arange(n)])
    deg = jnp.bincount(dst, length=n).astype(jnp.float32)
    dis = 1.0 / jnp.sqrt(deg)
    norm = (dis[src] * dis[dst]).astype(jnp.float32)
    h = emb[x]
    h1 = jax.ops.segment_sum(norm[:, None] * (h @ gcn1_W)[src], dst, num_segments=n) + gcn1_b
    h1 = jnp.maximum(_graph_norm(h1, gn1_w, gn1_b, gn1_m), 0.0)
    h2 = jax.ops.segment_sum(norm[:, None] * (h1 @ gcn2_W)[src], dst, num_segments=n) + gcn2_b
    h2 = jnp.maximum(_graph_norm(h2, gn2_w, gn2_b, gn2_m), 0.0)
    xx = h2[pos[:, 0]] * h2[pos[:, 1]]
    val = jnp.concatenate([h2[ei0], h2[ei1]], axis=1)
    D2 = m1_W.shape[1]
    xez = jnp.zeros((RPAD, D2), jnp.float32).at[:m].set(
        jnp.maximum(val @ m1_W + m1_b, 0.0))
    mulz = jnp.zeros((RPAD, D2), jnp.float32).at[:m].set(
        jnp.maximum(val @ m2_W + m2_b, 0.0))

    prod_s = lax.optimization_barrier(xez[ga]) * lax.optimization_barrier(mulz[gb])
    Cfull = jax.ops.segment_sum(prod_s, rank, num_segments=Eup,
                                indices_are_sorted=True)

    s1, s2 = _masked_stats(Cfull, onecol, Eu_n, m3_W, m3_b)
    den = Eu_n.astype(jnp.float32)
    mean3 = s1 / den
    mm = gn3_m * mean3
    var3 = s2 / den - 2.0 * mm * (s1 / den) + mm * mm

    rows = jnp.concatenate([predc, permp])
    Crows = Cfull[rows]
    orows = onecol[rows]
    x3r = Crows @ m3_W[:-1] + orows * m3_W[-1:] + m3_b
    x3r = jnp.maximum(gn3_w * (x3r - mm) / jnp.sqrt(var3 + 1e-5) + gn3_b, 0.0)
    npos = pos.shape[0]
    xp = jnp.where((pred < Eu)[:, None], x3r[:npos] * x3r[npos:], 0.0)
    out = jnp.concatenate([xp, xx], axis=1) @ dir_W + dir_b
    return out


# speedup vs baseline: 12.3434x; 1.1017x over previous
import functools

import jax
import jax.numpy as jnp
from jax import lax
from jax.experimental import pallas as pl
from jax.experimental.pallas import tpu as pltpu
from jax.experimental.pallas import tpu_sc as plsc

MAX_PAIRS = 3000000
BLK = 4000
SPAD = 2097152
RPAD = 1048576

_NC, _NS = 2, 16
_NW = _NC * _NS
_K = 1024
_SUB = 128
_NSUB = _K // _SUB
_CPT = 97
_NCHUNK = _NW * _CPT
_NPAD = _NCHUNK * _K
_D = 20
_DW = 32


def _graph_norm(x, w, b, ms, eps=1e-5):
    mean = jnp.mean(x, axis=0)
    xc = x - ms * mean
    var = jnp.mean(xc * xc, axis=0)
    return w * xc / jnp.sqrt(var + eps) + b


def _pad1(a, length, fill=0):
    return jnp.concatenate([a, jnp.full((length - a.shape[0],), fill, a.dtype)])


def _sc_scatter(xez, mulz, ga, gb, gl, glist2d, eup):
    mesh = plsc.VectorSubcoreMesh(core_axis_name="c", subcore_axis_name="s",
                                  num_cores=_NC, num_subcores=_NS)

    @functools.partial(
        pl.kernel,
        out_type=[jax.ShapeDtypeStruct((eup, _DW), jnp.float32),
                  jax.ShapeDtypeStruct((_NCHUNK, 1, _DW), jnp.float32)],
        mesh=mesh,
        compiler_params=pltpu.CompilerParams(use_tc_tiling_on_sc=False,
                                             needs_layout_passes=False),
        scratch_types=[
            pltpu.VMEM((_K,), jnp.int32),
            pltpu.VMEM((_K,), jnp.int32),
            pltpu.VMEM((_K,), jnp.int32),
            pltpu.VMEM((_NSUB, _SUB), jnp.int32),
            pltpu.VMEM((_K, _DW), jnp.float32),
            pltpu.VMEM((_K, _DW), jnp.float32),
            pltpu.VMEM((_K, _DW), jnp.float32),
            pltpu.SemaphoreType.DMA,
            pltpu.SemaphoreType.DMA,
            pltpu.SemaphoreType.DMA,
        ],
    )
    def k(xez_h, mulz_h, ga_h, gb_h, gl_h, gli_h, cout_h, side_h,
          ga_v, gb_v, gl_v, gli_v, ra_v, rb_v, cl_v, sem_a, sem_b, sem_c):
        wid = lax.axis_index("s") * _NC + lax.axis_index("c")
        iot = lax.iota(jnp.int32, 16)
        zz = jnp.zeros((16,), jnp.float32)

        def chunk(j, _):
            c = wid * _CPT + j
            base = c * _K
            pltpu.sync_copy(ga_h.at[pl.ds(base, _K)], ga_v)
            pltpu.sync_copy(gb_h.at[pl.ds(base, _K)], gb_v)
            pltpu.sync_copy(gl_h.at[pl.ds(base, _K)], gl_v)
            pltpu.sync_copy(gli_h.at[c], gli_v)
            handles = []
            for i in range(_NSUB):
                handles.append(pltpu.async_copy(
                    xez_h.at[ga_v.at[pl.ds(i * _SUB, _SUB)]],
                    ra_v.at[pl.ds(i * _SUB, _SUB)], sem_a))
                handles.append(pltpu.async_copy(
                    mulz_h.at[gb_v.at[pl.ds(i * _SUB, _SUB)]],
                    rb_v.at[pl.ds(i * _SUB, _SUB)], sem_b))

            def zrow(g, _):
                rv = g * 16 + iot
                for ch in range(_DW):
                    plsc.store_scatter(cl_v, [rv, jnp.full((16,), ch, jnp.int32)], zz)
                return 0

            lax.fori_loop(0, _K // 16, zrow, 0, unroll=False)
            for h in handles:
                h.wait()

            def acc(g, _):
                rv = g * 16 + iot
                lid = gl_v[pl.ds(g * 16, 16)]
                for ch in range(_D):
                    chv = jnp.full((16,), ch, jnp.int32)
                    av = plsc.load_gather(ra_v, [rv, chv])
                    bv = plsc.load_gather(rb_v, [rv, chv])
                    plsc.addupdate_scatter(cl_v, [lid, chv], av * bv)
                return 0

            lax.fori_loop(0, _K // 16, acc, 0, unroll=False)
            whandles = []
            for i in range(_NSUB):
                whandles.append(pltpu.async_copy(
                    cl_v.at[pl.ds(i * _SUB, _SUB)],
                    cout_h.at[gli_v.at[i]], sem_c))
            whandles.append(pltpu.async_copy(
                cl_v.at[pl.ds(0, 1)], side_h.at[c], sem_c))
            for h in whandles:
                h.wait()
            return 0

        lax.fori_loop(0, _CPT, chunk, 0, unroll=False)

    return k(xez, mulz, ga, gb, gl, glist2d)


def _sc_cfull(xez, mulz, ga, gb, rank, eup):
    m_zero = xez.shape[0] - 1
    trash = eup - 1
    Eu = rank.shape[0]
    ga_p = _pad1(ga, _NPAD, m_zero)
    gb_p = _pad1(gb, _NPAD, m_zero)
    rank_p = _pad1(rank, _NPAD, 0)
    cidx = jnp.arange(_NCHUNK, dtype=jnp.int32)
    real = cidx * _K < Eu
    fr = jnp.where(real, rank[jnp.minimum(cidx * _K, Eu - 1)], trash)
    lastr = rank[jnp.minimum((cidx + 1) * _K - 1, Eu - 1)]
    nloc = jnp.where(real, lastr - fr + 1, 1)
    frent = jnp.broadcast_to(fr[:, None], (_NCHUNK, _K)).reshape(-1)
    gl = jnp.clip(rank_p - frent, 0, _K - 1)
    l = jnp.arange(_K, dtype=jnp.int32)[None, :]
    glist = jnp.where((l >= 1) & (l <= (nloc - 1)[:, None]) & real[:, None],
                      fr[:, None] + l, trash).reshape(_NCHUNK, _NSUB, _SUB)
    cout, side = _sc_scatter(xez, mulz, ga_p, gb_p, gl, glist, eup)
    side = side.reshape(_NCHUNK, _DW)
    covend = lax.cummax(jnp.where(real & (nloc >= 2), fr + nloc - 1, -1))
    covend_prev = jnp.concatenate([jnp.full((1,), -1, covend.dtype), covend[:-1]])
    fr_prev = jnp.concatenate([jnp.full((1,), -1, fr.dtype), fr[:-1]])
    uncov = real & (fr > covend_prev)
    firstuncov = uncov & ((cidx == 0) | (fr != fr_prev))
    fr_set = jnp.where(firstuncov, fr, trash)
    fr_add = jnp.where(real & ~firstuncov, fr, trash)
    return cout.at[fr_set].set(side).at[fr_add].add(side)


def _stats_body(eun_ref, c_ref, one_ref, w3_ref, wrow_ref, b3_ref, s1_ref, s2_ref):
    pid = pl.program_id(0)
    R = c_ref.shape[0]
    x3 = (jnp.dot(c_ref[...], w3_ref[...], preferred_element_type=jnp.float32)
          + one_ref[...] * wrow_ref[...] + b3_ref[...])
    gidx = pid * R + lax.broadcasted_iota(jnp.int32, (R, 1), 0)
    rowm = gidx < eun_ref[0]
    x3m = jnp.where(rowm, x3, 0.0)
    s1 = jnp.sum(x3m, axis=0, keepdims=True)
    s2 = jnp.sum(x3m * x3m, axis=0, keepdims=True)

    @pl.when(pid == 0)
    def _():
        s1_ref[...] = jnp.zeros_like(s1_ref)
        s2_ref[...] = jnp.zeros_like(s2_ref)

    s1_ref[...] += s1
    s2_ref[...] += s2


def _masked_stats(Cfull, onecol, Eu_n, m3_W, m3_b):
    Eup, DW = Cfull.shape
    D = m3_W.shape[1]
    w32 = jnp.zeros((DW, D), jnp.float32).at[:D].set(m3_W[:D])
    grid = Eup // BLK
    s1, s2 = pl.pallas_call(
        _stats_body,
        grid=(grid,),
        in_specs=[
            pl.BlockSpec(memory_space=pltpu.SMEM),
            pl.BlockSpec((BLK, DW), lambda i: (i, 0)),
            pl.BlockSpec((BLK, 1), lambda i: (i, 0)),
            pl.BlockSpec((DW, D), lambda i: (0, 0)),
            pl.BlockSpec((1, D), lambda i: (0, 0)),
            pl.BlockSpec((1, D), lambda i: (0, 0)),
        ],
        out_specs=[
            pl.BlockSpec((1, D), lambda i: (0, 0)),
            pl.BlockSpec((1, D), lambda i: (0, 0)),
        ],
        out_shape=[
            jax.ShapeDtypeStruct((1, D), jnp.float32),
            jax.ShapeDtypeStruct((1, D), jnp.float32),
        ],
    )(
        Eu_n.reshape(1).astype(jnp.int32),
        Cfull,
        onecol,
        w32,
        m3_W[D:D + 1],
        m3_b.reshape(1, D),
    )
    return s1[0], s2[0]


def kernel(x, ei, pos, emb, gcn1_W, gcn1_b, gn1_w, gn1_b, gn1_m, gcn2_W, gcn2_b,
           gn2_w, gn2_b, gn2_m, m1_W, m1_b, m2_W, m2_b, m3_W, m3_b, gn3_w,
           gn3_b, gn3_m, dir_W, dir_b):
    n = x.shape[0]
    m = ei.shape[1]
    sent = n * n
    Eu = MAX_PAIRS + m
    Eup = ((Eu + BLK) // BLK) * BLK

    ei0 = ei[0]
    ei1 = ei[1]
    order = jnp.argsort(ei0, stable=True).astype(jnp.int32)
    counts = jnp.bincount(ei0, length=n)
    group_start = (jnp.cumsum(counts) - counts).astype(jnp.int32)
    rep = counts[ei1]
    total = rep.sum()
    cum = (jnp.cumsum(rep) - rep).astype(jnp.int32)
    z = jnp.full((MAX_PAIRS,), -1, jnp.int32).at[cum].max(
        jnp.arange(m, dtype=jnp.int32), mode='drop')
    a_idx = lax.cummax(z)
    delta_e = _pad1(group_start[ei1] - cum, SPAD)
    i_n_e = _pad1((ei0 * n).astype(jnp.int32), SPAD)
    packedord = _pad1(order * 10000 + ei1[order], SPAD)
    parange = jnp.arange(MAX_PAIRS, dtype=jnp.int32)
    b_pos = jnp.clip(delta_e[a_idx] + parange, 0, m - 1)
    pko = packedord[b_pos]
    b_idx = pko // 10000
    jdst = pko - b_idx * 10000
    valid = parange < total
    key = jnp.where(valid, i_n_e[a_idx] + jdst, sent)

    keys_e = ei0 * n + ei1
    allkeys = jnp.concatenate([key, keys_e])
    ga_all = jnp.concatenate([jnp.where(valid, a_idx, m),
                              jnp.full((m,), m, jnp.int32)])
    gb_all = jnp.concatenate([jnp.where(valid, b_idx, m),
                              jnp.full((m,), m, jnp.int32)])
    sp = jnp.argsort(allkeys, stable=True).astype(jnp.int32)
    ks = allkeys[sp]
    ga = ga_all[sp]
    gb = gb_all[sp]
    newk = jnp.concatenate([jnp.ones((1,), jnp.int32),
                            (ks[1:] != ks[:-1]).astype(jnp.int32)])
    rank = jnp.cumsum(newk) - 1
    n_uniq = rank[-1] + 1
    Eu_n = n_uniq - (ks[-1] == sent).astype(jnp.int32)

    pos_e = jnp.searchsorted(ks, keys_e).astype(jnp.int32)
    rank_e = rank[pos_e]
    onecol = jnp.zeros((Eup,), jnp.float32).at[rank_e].add(1.0)[:, None]

    key_p = pos[:, 0] * n + pos[:, 1]
    qpos = jnp.minimum(jnp.searchsorted(ks, key_p).astype(jnp.int32), Eu - 1)
    qhit = ks[qpos] == key_p
    pred = jnp.where(qhit, rank[qpos], Eu)

    tkeys = jnp.where(allkeys < sent, (allkeys % n) * n + allkeys // n, sent)
    tsp = jnp.argsort(tkeys, stable=True).astype(jnp.int32)
    tks = tkeys[tsp]
    tnew = jnp.concatenate([jnp.ones((1,), jnp.int32),
                            (tks[1:] != tks[:-1]).astype(jnp.int32)])
    trank = jnp.cumsum(tnew) - 1
    predc = jnp.minimum(pred, Eu - 1)
    tfirst = jnp.minimum(jnp.searchsorted(trank, predc).astype(jnp.int32), Eu - 1)
    partner_key = allkeys[tsp[tfirst]]
    ppos = jnp.minimum(jnp.searchsorted(ks, partner_key).astype(jnp.int32), Eu - 1)
    permp = rank[ppos]

    src = jnp.concatenate([ei0, jnp.arange(n)])
    dst = jnp.concatenate([ei1, jnp.arange(n)])
    deg = jnp.bincount(dst, length=n).astype(jnp.float32)
    dis = 1.0 / jnp.sqrt(deg)
    norm = (dis[src] * dis[dst]).astype(jnp.float32)
    h = emb[x]
    h1 = jax.ops.segment_sum(norm[:, None] * (h @ gcn1_W)[src], dst, num_segments=n) + gcn1_b
    h1 = jnp.maximum(_graph_norm(h1, gn1_w, gn1_b, gn1_m), 0.0)
    h2 = jax.ops.segment_sum(norm[:, None] * (h1 @ gcn2_W)[src], dst, num_segments=n) + gcn2_b
    h2 = jnp.maximum(_graph_norm(h2, gn2_w, gn2_b, gn2_m), 0.0)
    xx = h2[pos[:, 0]] * h2[pos[:, 1]]
    val = jnp.concatenate([h2[ei0], h2[ei1]], axis=1)
    xez = jnp.zeros((RPAD, _DW), jnp.float32).at[:m, :_D].set(
        jnp.maximum(val @ m1_W + m1_b, 0.0))
    mulz = jnp.zeros((RPAD, _DW), jnp.float32).at[:m, :_D].set(
        jnp.maximum(val @ m2_W + m2_b, 0.0))

    Cfull = _sc_cfull(xez, mulz, ga, gb, rank.astype(jnp.int32), Eup)

    s1, s2 = _masked_stats(Cfull, onecol, Eu_n, m3_W, m3_b)
    den = Eu_n.astype(jnp.float32)
    mean3 = s1 / den
    mm = gn3_m * mean3
    var3 = s2 / den - 2.0 * mm * (s1 / den) + mm * mm

    rows = jnp.concatenate([predc, permp])
    Crows = Cfull[rows]
    orows = onecol[rows]
    w32t = jnp.zeros((_DW, _D), jnp.float32).at[:_D].set(m3_W[:-1])
    x3r = Crows @ w32t + orows * m3_W[-1:] + m3_b
    x3r = jnp.maximum(gn3_w * (x3r - mm) / jnp.sqrt(var3 + 1e-5) + gn3_b, 0.0)
    npos = pos.shape[0]
    xp = jnp.where((pred < Eu)[:, None], x3r[:npos] * x3r[npos:], 0.0)
    out = jnp.concatenate([xp, xx], axis=1) @ dir_W + dir_b
    return out
